# baseline (speedup 1.0000x reference)
import functools

import jax
import jax.numpy as jnp
from jax import lax
from jax.experimental import pallas as pl
from jax.experimental.pallas import tpu as pltpu
from jax.experimental.pallas import tpu_sc as plsc

N = 50000
E = 800000
FEAT = 128

NC, NS, L = 2, 16, 16
NW = NC * NS
EW = 25600
EPAD = NW * EW
CHUNK = 1280
NCHUNK = EW // CHUNK
NSUB = CHUNK // 128
NVREG = CHUNK // L
HALF = 25088
NPAD = 2 * HALF
TROWS = HALF // NS
TWORDS = TROWS * 4
HWORDS = HALF * 4


def _dense_body(xf_ref, xfb_ref, xt_ref, pf_ref, pfb_ref, q_ref, r_ref,
                c_ref, bv_ref, base_ref):
    xf = xf_ref[...]
    xfb = xfb_ref[...]
    xt = xt_ref[...]
    spa4 = xt[:, 0:4]
    last = xt[:, 4:5]
    idx = jnp.minimum(last - 1.0, 2.0).astype(jnp.int32)
    hi = jax.lax.Precision.HIGHEST
    out6 = jnp.dot(xf, pf_ref[...], preferred_element_type=jnp.float32,
                   precision=hi)
    out6 += jnp.dot(xfb, pfb_ref[...], preferred_element_type=jnp.float32,
                    precision=hi)
    out6 += jnp.dot(spa4, r_ref[...], preferred_element_type=jnp.float32,
                    precision=hi)
    for k in range(3):
        out6 += jnp.where(idx == k, 1.0, 0.0) * q_ref[k:k + 1, :]
    out6 += c_ref[...]
    bv_ref[...] = out6[:, 0:4]
    base_ref[...] = out6[:, 4:6]


def _dense_call(x, pf, pfb, q, r, c):
    bn = 1024
    grid = (pl.cdiv(N, bn),)
    return pl.pallas_call(
        _dense_body,
        grid=grid,
        in_specs=[
            pl.BlockSpec((bn, FEAT), lambda i: (i, 0)),
            pl.BlockSpec((bn, FEAT), lambda i: (i, 1)),
            pl.BlockSpec((bn, FEAT), lambda i: (i, 7)),
            pl.BlockSpec((FEAT, 6), lambda i: (0, 0)),
            pl.BlockSpec((FEAT, 6), lambda i: (0, 0)),
            pl.BlockSpec((3, 6), lambda i: (0, 0)),
            pl.BlockSpec((4, 6), lambda i: (0, 0)),
            pl.BlockSpec((1, 6), lambda i: (0, 0)),
        ],
        out_specs=[
            pl.BlockSpec((bn, 4), lambda i: (i, 0)),
            pl.BlockSpec((bn, 2), lambda i: (i, 0)),
        ],
        out_shape=[
            jax.ShapeDtypeStruct((N, 4), jnp.float32),
            jax.ShapeDtypeStruct((N, 2), jnp.float32),
        ],
    )(x, x, x, pf, pfb, q, r, c)


def _edge_body(dst_ref, attr_ref, dstm_ref):
    i = pl.program_id(0)
    d = dst_ref[...]
    a = attr_ref[...]
    rows = lax.broadcasted_iota(jnp.int32, d.shape, 0) + i * d.shape[0]
    cols = lax.broadcasted_iota(jnp.int32, d.shape, 1)
    e = rows * 128 + cols
    keep = (e < E) & ((a == 0) | (a == 111))
    dstm_ref[...] = jnp.where(keep, d, N)


def _edge_call(dst2d, attr2d):
    bn = 256
    rows_out = EPAD // 128
    grid = (rows_out // bn,)
    return pl.pallas_call(
        _edge_body,
        grid=grid,
        in_specs=[
            pl.BlockSpec((bn, 128), lambda i: (i, 0)),
            pl.BlockSpec((bn, 128), lambda i: (i, 0)),
        ],
        out_specs=pl.BlockSpec((bn, 128), lambda i: (i, 0)),
        out_shape=jax.ShapeDtypeStruct((rows_out, 128), jnp.int32),
    )(dst2d, attr2d)


def _sc_body(src_hbm, dstm_hbm, bvflat_hbm, out_hbm,
             srcv, dstv, idx1, idx2, idx3,
             vals0, vals1, vals2, vals3, maxtab, sem):
    c = lax.axis_index("c")
    s = lax.axis_index("s")
    w = s * NC + c
    ebase = w * EW
    neginf = jnp.full((L,), -jnp.inf, dtype=jnp.float32)
    idxrefs = [srcv, idx1, idx2, idx3]
    valrefs = [vals0, vals1, vals2, vals3]

    for p in range(2):
        lo = p * HALF
        hi = min((p + 1) * HALF, N)

        def init_body(k, _):
            maxtab[pl.ds(k * L, L)] = neginf
            return _
        lax.fori_loop(0, HWORDS // L, init_body, None)

        def chunk_body(ch, _):
            cb = ebase + ch * CHUNK
            pltpu.sync_copy(src_hbm.at[pl.ds(cb, CHUNK)], srcv)
            pltpu.sync_copy(dstm_hbm.at[pl.ds(cb, CHUNK)], dstv)

            def idx_body(v, _):
                sl = pl.ds(v * L, L)
                s16 = srcv[sl]
                idx1[sl] = s16 + N
                idx2[sl] = s16 + 2 * N
                idx3[sl] = s16 + 3 * N
                return _
            lax.fori_loop(0, NVREG, idx_body, None)

            handles = [
                pltpu.async_copy(
                    bvflat_hbm.at[idxrefs[j].at[pl.ds(sub * 128, 128)]],
                    valrefs[j].at[pl.ds(sub * 128, 128)], sem)
                for j in range(4) for sub in range(NSUB)
            ]
            for h in handles:
                h.wait()

            def vreg_body(v, _):
                sl = pl.ds(v * L, L)
                dst = dstv[sl]
                m = (dst >= lo) & (dst < hi)
                local4 = jnp.where(m, (dst - lo) * 4, 0)
                vj = [vr[sl] for vr in valrefs]

                def fix_cond(act):
                    return jnp.any(act)

                def fix_body(act):
                    newact = jnp.zeros((L,), dtype=jnp.bool_)
                    for j in range(4):
                        old = plsc.load_gather(maxtab, [local4 + j], mask=act)
                        wr = act & (vj[j] > old)
                        plsc.store_scatter(maxtab, [local4 + j], vj[j], mask=wr)
                        newact = newact | wr
                    return newact

                lax.while_loop(fix_cond, fix_body, m)
                return _
            lax.fori_loop(0, NVREG, vreg_body, None)
            return _
        lax.fori_loop(0, NCHUNK, chunk_body, None)

        pltpu.sync_copy(maxtab, out_hbm.at[w, pl.ds(p * HWORDS, HWORDS)])


@functools.cache
def _sc_call():
    return pl.kernel(
        _sc_body,
        out_type=jax.ShapeDtypeStruct((NW, 2 * HWORDS), jnp.float32),
        mesh=plsc.VectorSubcoreMesh(core_axis_name="c", subcore_axis_name="s",
                                    num_cores=NC, num_subcores=NS),
        compiler_params=pltpu.CompilerParams(needs_layout_passes=False),
        scratch_types=(
            [pltpu.VMEM((CHUNK,), jnp.int32) for _ in range(5)]
            + [pltpu.VMEM((CHUNK,), jnp.float32) for _ in range(4)]
            + [
                pltpu.VMEM((HWORDS,), jnp.float32),
                pltpu.SemaphoreType.DMA,
            ]
        ),
    )


def _reduce_body(p_ref, mx_ref):
    mx_ref[...] = jnp.max(p_ref[...], axis=0)


def _reduce_call(partials):
    rows = NPAD * 4 // 128
    bm = 224
    grid = (rows // bm,)
    return pl.pallas_call(
        _reduce_body,
        grid=grid,
        in_specs=[pl.BlockSpec((NW, bm, 128), lambda i: (0, i, 0))],
        out_specs=pl.BlockSpec((bm, 128), lambda i: (i, 0)),
        out_shape=jax.ShapeDtypeStruct((rows, 128), jnp.float32),
    )(partials)


def _combine_body(mx_ref, base_ref, out_ref):
    mx = mx_ref[...]
    empty = mx[:, 0:1] == -jnp.inf
    val = base_ref[...] + mx[:, 0:2] + mx[:, 2:4]
    out_ref[...] = jnp.where(empty, 0.0, val)


def _combine_call(mx, base):
    bn = 2048
    grid = (pl.cdiv(N, bn),)
    return pl.pallas_call(
        _combine_body,
        grid=grid,
        in_specs=[
            pl.BlockSpec((bn, 4), lambda i: (i, 0)),
            pl.BlockSpec((bn, 2), lambda i: (i, 0)),
        ],
        out_specs=pl.BlockSpec((bn, 2), lambda i: (i, 0)),
        out_shape=jax.ShapeDtypeStruct((N, 2), jnp.float32),
    )(mx, base)


def kernel(x, edge_index, edge_attr, spk_W, spk_b, spa_W, spa_b,
           f_fc1_W, f_fc1_b, f_fc2_W, f_fc2_b, f_mlp_W, f_mlp_b,
           fb_fc1_W, fb_fc1_b, fb_fc2_W, fb_fc2_b, fb_mlp_W, fb_mlp_b):
    def mm(a, b):
        return jnp.dot(a, b, precision=jax.lax.Precision.HIGHEST,
                       preferred_element_type=jnp.float32)

    def fold(fc1_W, fc1_b, fc2_W, fc2_b, mlp_W):
        W = fc1_W + fc2_W[:FEAT]
        S = mm(spk_W, fc2_W[FEAT:FEAT + 16])
        A = mm(spa_W, fc2_W[FEAT + 16:FEAT + 32])
        cv = fc1_b + fc2_b + mm(spk_b, fc2_W[FEAT:FEAT + 16]) \
            + mm(spa_b, fc2_W[FEAT + 16:FEAT + 32])
        G = mlp_W[:64] - mlp_W[64:]
        B = mlp_W[64:]
        return W, S, A, cv, G, B

    Wf, Sf, Af, cf, Gf, Bf = fold(f_fc1_W, f_fc1_b, f_fc2_W, f_fc2_b, f_mlp_W)
    Wb, Sb, Ab, cb, Gb, Bb = fold(fb_fc1_W, fb_fc1_b, fb_fc2_W, fb_fc2_b,
                                  fb_mlp_W)
    z2 = jnp.zeros((FEAT, 2), jnp.float32)
    pf = jnp.concatenate([mm(Wf, Bf), z2, mm(Wf, Gf)], axis=1)
    pfb = jnp.concatenate([z2, mm(Wb, Bb), mm(Wb, Gb)], axis=1)
    q = jnp.concatenate([mm(Sf, Bf), mm(Sb, Bb), mm(Sf, Gf) + mm(Sb, Gb)],
                        axis=1)
    r = jnp.concatenate([mm(Af, Bf), mm(Ab, Bb), mm(Af, Gf) + mm(Ab, Gb)],
                        axis=1)
    c6 = jnp.concatenate([mm(cf, Bf), mm(cb, Bb),
                          mm(cf, Gf) + mm(cb, Gb) + f_mlp_b + fb_mlp_b])[None, :]

    bv4, base = _dense_call(x, pf, pfb, q, r, c6)

    src = edge_index[0]
    dst = edge_index[1]
    pad = jnp.arange(EPAD - E, dtype=jnp.int32) % N
    src_pad = jnp.concatenate([src, pad])
    dst2d = jnp.concatenate([dst, pad]).reshape(EPAD // 128, 128)
    attr2d = jnp.concatenate(
        [edge_attr, jnp.ones((EPAD - E,), jnp.int32)]).reshape(EPAD // 128, 128)
    dstm = _edge_call(dst2d, attr2d).reshape(EPAD)

    bvflat = bv4.T.reshape(4 * N)
    partials = _sc_call()(src_pad, dstm, bvflat).reshape(NW, NPAD * 4 // 128,
                                                         128)
    mx = _reduce_call(partials).reshape(NPAD, 4)
    return _combine_call(mx, base)


# speedup vs baseline: 1.1286x; 1.1286x over previous
import functools

import jax
import jax.numpy as jnp
from jax import lax
from jax.experimental import pallas as pl
from jax.experimental.pallas import tpu as pltpu
from jax.experimental.pallas import tpu_sc as plsc

N = 50000
E = 800000
FEAT = 128

NC, NS, L = 2, 16, 16
NW = NC * NS
EW = 25600
EPAD = NW * EW
CHUNK = 1280
NCHUNK = EW // CHUNK
NSUB = CHUNK // 128
NVREG = CHUNK // L
HALF = 25088
NPAD = 2 * HALF
TROWS = HALF // NS
TWORDS = TROWS * 4
HWORDS = HALF * 4


def _dense_body(xf_ref, xfb_ref, xt_ref, pf_ref, pfb_ref, q_ref, r_ref,
                c_ref, bv_ref, base_ref):
    xf = xf_ref[...]
    xfb = xfb_ref[...]
    xt = xt_ref[...]
    spa4 = xt[:, 0:4]
    last = xt[:, 4:5]
    idx = jnp.minimum(last - 1.0, 2.0).astype(jnp.int32)
    hi = jax.lax.Precision.HIGHEST
    out6 = jnp.dot(xf, pf_ref[...], preferred_element_type=jnp.float32,
                   precision=hi)
    out6 += jnp.dot(xfb, pfb_ref[...], preferred_element_type=jnp.float32,
                    precision=hi)
    out6 += jnp.dot(spa4, r_ref[...], preferred_element_type=jnp.float32,
                    precision=hi)
    for k in range(3):
        out6 += jnp.where(idx == k, 1.0, 0.0) * q_ref[k:k + 1, :]
    out6 += c_ref[...]
    bv_ref[...] = out6[:, 0:4]
    base_ref[...] = out6[:, 4:6]


def _dense_call(x, pf, pfb, q, r, c):
    bn = 1024
    grid = (pl.cdiv(N, bn),)
    return pl.pallas_call(
        _dense_body,
        grid=grid,
        in_specs=[
            pl.BlockSpec((bn, FEAT), lambda i: (i, 0)),
            pl.BlockSpec((bn, FEAT), lambda i: (i, 1)),
            pl.BlockSpec((bn, FEAT), lambda i: (i, 7)),
            pl.BlockSpec((FEAT, 6), lambda i: (0, 0)),
            pl.BlockSpec((FEAT, 6), lambda i: (0, 0)),
            pl.BlockSpec((3, 6), lambda i: (0, 0)),
            pl.BlockSpec((4, 6), lambda i: (0, 0)),
            pl.BlockSpec((1, 6), lambda i: (0, 0)),
        ],
        out_specs=[
            pl.BlockSpec((bn, 4), lambda i: (i, 0)),
            pl.BlockSpec((bn, 2), lambda i: (i, 0)),
        ],
        out_shape=[
            jax.ShapeDtypeStruct((N, 4), jnp.float32),
            jax.ShapeDtypeStruct((N, 2), jnp.float32),
        ],
    )(x, x, x, pf, pfb, q, r, c)


def _edge_body(dst_ref, attr_ref, dstm_ref):
    i = pl.program_id(0)
    d = dst_ref[...]
    a = attr_ref[...]
    rows = lax.broadcasted_iota(jnp.int32, d.shape, 0) + i * d.shape[0]
    cols = lax.broadcasted_iota(jnp.int32, d.shape, 1)
    e = rows * 128 + cols
    keep = (e < E) & ((a == 0) | (a == 111))
    dstm_ref[...] = jnp.where(keep, d, N)


def _edge_call(dst2d, attr2d):
    bn = 256
    rows_out = EPAD // 128
    grid = (rows_out // bn,)
    return pl.pallas_call(
        _edge_body,
        grid=grid,
        in_specs=[
            pl.BlockSpec((bn, 128), lambda i: (i, 0)),
            pl.BlockSpec((bn, 128), lambda i: (i, 0)),
        ],
        out_specs=pl.BlockSpec((bn, 128), lambda i: (i, 0)),
        out_shape=jax.ShapeDtypeStruct((rows_out, 128), jnp.int32),
    )(dst2d, attr2d)


def _sc_body(src_hbm, dstm_hbm, bvflat_hbm, out_hbm,
             srcv, dstv, ia, ib, va, vb, maxtab, sem):
    c = lax.axis_index("c")
    s = lax.axis_index("s")
    w = s * NC + c
    ebase = w * EW
    neginf = jnp.full((L,), -jnp.inf, dtype=jnp.float32)

    for p in range(2):
        offA = (2 * p) * N
        offB = (2 * p + 1) * N

        def init_body(k, _):
            maxtab[pl.ds(k * L, L)] = neginf
            return _
        lax.fori_loop(0, 2 * NPAD // L, init_body, None)

        def chunk_body(ch, _):
            cb = ebase + ch * CHUNK
            pltpu.sync_copy(src_hbm.at[pl.ds(cb, CHUNK)], srcv)
            pltpu.sync_copy(dstm_hbm.at[pl.ds(cb, CHUNK)], dstv)

            def idx_body(v, _):
                sl = pl.ds(v * L, L)
                s16 = srcv[sl]
                ia[sl] = s16 + offA
                ib[sl] = s16 + offB
                return _
            lax.fori_loop(0, NVREG, idx_body, None)

            h1 = pltpu.async_copy(bvflat_hbm.at[ia], va, sem)
            h2 = pltpu.async_copy(bvflat_hbm.at[ib], vb, sem)
            h1.wait()
            h2.wait()

            def vreg_body(v, _):
                sl = pl.ds(v * L, L)
                dst = dstv[sl]
                m = dst < N
                loc2 = jnp.where(m, dst * 2, 0)
                vA = va[sl]
                vB = vb[sl]

                def fix_cond(act):
                    return jnp.any(act)

                def fix_body(act):
                    oldA = plsc.load_gather(maxtab, [loc2], mask=act)
                    wrA = act & (vA > oldA)
                    plsc.store_scatter(maxtab, [loc2], vA, mask=wrA)
                    oldB = plsc.load_gather(maxtab, [loc2 + 1], mask=act)
                    wrB = act & (vB > oldB)
                    plsc.store_scatter(maxtab, [loc2 + 1], vB, mask=wrB)
                    return wrA | wrB

                lax.while_loop(fix_cond, fix_body, m)
                return _
            lax.fori_loop(0, NVREG, vreg_body, None)
            return _
        lax.fori_loop(0, NCHUNK, chunk_body, None)

        pltpu.sync_copy(maxtab, out_hbm.at[w, pl.ds(p * 2 * NPAD, 2 * NPAD)])


@functools.cache
def _sc_call():
    return pl.kernel(
        _sc_body,
        out_type=jax.ShapeDtypeStruct((NW, 2 * HWORDS), jnp.float32),
        mesh=plsc.VectorSubcoreMesh(core_axis_name="c", subcore_axis_name="s",
                                    num_cores=NC, num_subcores=NS),
        compiler_params=pltpu.CompilerParams(needs_layout_passes=False),
        scratch_types=(
            [pltpu.VMEM((CHUNK,), jnp.int32) for _ in range(4)]
            + [pltpu.VMEM((CHUNK,), jnp.float32) for _ in range(2)]
            + [
                pltpu.VMEM((2 * NPAD,), jnp.float32),
                pltpu.SemaphoreType.DMA,
            ]
        ),
    )


def _reduce_body(p_ref, mx_ref):
    mx_ref[...] = jnp.max(p_ref[...], axis=0)


def _reduce_call(partials):
    rows = NPAD * 4 // 128
    bm = 224
    grid = (rows // bm,)
    return pl.pallas_call(
        _reduce_body,
        grid=grid,
        in_specs=[pl.BlockSpec((NW, bm, 128), lambda i: (0, i, 0))],
        out_specs=pl.BlockSpec((bm, 128), lambda i: (i, 0)),
        out_shape=jax.ShapeDtypeStruct((rows, 128), jnp.float32),
    )(partials)


def _combine_body(mx01_ref, mx23_ref, base_ref, out_ref):
    mx01 = mx01_ref[...]
    mx23 = mx23_ref[...]
    empty = mx01[:, 0:1] == -jnp.inf
    val = base_ref[...] + mx01 + mx23
    out_ref[...] = jnp.where(empty, 0.0, val)


def _combine_call(mx01, mx23, base):
    bn = 2048
    grid = (pl.cdiv(N, bn),)
    return pl.pallas_call(
        _combine_body,
        grid=grid,
        in_specs=[
            pl.BlockSpec((bn, 2), lambda i: (i, 0)),
            pl.BlockSpec((bn, 2), lambda i: (i, 0)),
            pl.BlockSpec((bn, 2), lambda i: (i, 0)),
        ],
        out_specs=pl.BlockSpec((bn, 2), lambda i: (i, 0)),
        out_shape=jax.ShapeDtypeStruct((N, 2), jnp.float32),
    )(mx01, mx23, base)


def kernel(x, edge_index, edge_attr, spk_W, spk_b, spa_W, spa_b,
           f_fc1_W, f_fc1_b, f_fc2_W, f_fc2_b, f_mlp_W, f_mlp_b,
           fb_fc1_W, fb_fc1_b, fb_fc2_W, fb_fc2_b, fb_mlp_W, fb_mlp_b):
    def mm(a, b):
        return jnp.dot(a, b, precision=jax.lax.Precision.HIGHEST,
                       preferred_element_type=jnp.float32)

    def fold(fc1_W, fc1_b, fc2_W, fc2_b, mlp_W):
        W = fc1_W + fc2_W[:FEAT]
        S = mm(spk_W, fc2_W[FEAT:FEAT + 16])
        A = mm(spa_W, fc2_W[FEAT + 16:FEAT + 32])
        cv = fc1_b + fc2_b + mm(spk_b, fc2_W[FEAT:FEAT + 16]) \
            + mm(spa_b, fc2_W[FEAT + 16:FEAT + 32])
        G = mlp_W[:64] - mlp_W[64:]
        B = mlp_W[64:]
        return W, S, A, cv, G, B

    Wf, Sf, Af, cf, Gf, Bf = fold(f_fc1_W, f_fc1_b, f_fc2_W, f_fc2_b, f_mlp_W)
    Wb, Sb, Ab, cb, Gb, Bb = fold(fb_fc1_W, fb_fc1_b, fb_fc2_W, fb_fc2_b,
                                  fb_mlp_W)
    z2 = jnp.zeros((FEAT, 2), jnp.float32)
    pf = jnp.concatenate([mm(Wf, Bf), z2, mm(Wf, Gf)], axis=1)
    pfb = jnp.concatenate([z2, mm(Wb, Bb), mm(Wb, Gb)], axis=1)
    q = jnp.concatenate([mm(Sf, Bf), mm(Sb, Bb), mm(Sf, Gf) + mm(Sb, Gb)],
                        axis=1)
    r = jnp.concatenate([mm(Af, Bf), mm(Ab, Bb), mm(Af, Gf) + mm(Ab, Gb)],
                        axis=1)
    c6 = jnp.concatenate([mm(cf, Bf), mm(cb, Bb),
                          mm(cf, Gf) + mm(cb, Gb) + f_mlp_b + fb_mlp_b])[None, :]

    bv4, base = _dense_call(x, pf, pfb, q, r, c6)

    src = edge_index[0]
    dst = edge_index[1]
    pad = jnp.arange(EPAD - E, dtype=jnp.int32) % N
    src_pad = jnp.concatenate([src, pad])
    dst2d = jnp.concatenate([dst, pad]).reshape(EPAD // 128, 128)
    attr2d = jnp.concatenate(
        [edge_attr, jnp.ones((EPAD - E,), jnp.int32)]).reshape(EPAD // 128, 128)
    dstm = _edge_call(dst2d, attr2d).reshape(EPAD)

    bvflat = bv4.T.reshape(4 * N)
    partials = _sc_call()(src_pad, dstm, bvflat).reshape(NW, NPAD * 4 // 128,
                                                         128)
    mx = _reduce_call(partials).reshape(2, NPAD, 2)
    return _combine_call(mx[0], mx[1], base)
